# baseline (speedup 1.0000x reference)
import functools

import jax
import jax.numpy as jnp
from jax import lax
from jax.experimental import pallas as pl
from jax.experimental.pallas import tpu as pltpu
from jax.experimental.pallas import tpu_sc as plsc

_NUM_EMB = 256
_EMB = 3

_info = plsc.get_sparse_core_info()
_NC, _NS, _L = _info.num_cores, _info.num_subcores, _info.num_lanes
_NW = _NC * _NS


def _min_body(x_ref, o_ref):
    @pl.when(pl.program_id(0) == 0)
    def _():
        o_ref[...] = jnp.full_like(o_ref[...], jnp.inf)

    o_ref[...] = jnp.minimum(o_ref[...], jnp.min(x_ref[...]))


def _global_min(feature):
    B, C, H, Wd = feature.shape
    grid = 32
    bb = B // grid
    out = pl.pallas_call(
        _min_body,
        grid=(grid,),
        in_specs=[pl.BlockSpec((bb, C, H, Wd), lambda i: (i, 0, 0, 0))],
        out_specs=pl.BlockSpec((8, 128), lambda i: (0, 0)),
        out_shape=jax.ShapeDtypeStruct((8, 128), jnp.float32),
    )(feature)
    return out


def _sc_lookup(feature, w_flat, m2d, planes_per_tile, rows):
    B, C, H, Wd = feature.shape
    piece = rows * Wd
    pieces_per_plane = H // rows
    nsteps = planes_per_tile * pieces_per_plane
    assert nsteps % 2 == 0
    mesh = plsc.VectorSubcoreMesh(core_axis_name="c", subcore_axis_name="s")

    @functools.partial(
        pl.kernel,
        mesh=mesh,
        compiler_params=pltpu.CompilerParams(needs_layout_passes=False),
        out_type=jax.ShapeDtypeStruct((B, _EMB, H, Wd), jnp.float32),
        scratch_types=[
            pltpu.VMEM((_EMB * _NUM_EMB,), jnp.float32),
            pltpu.VMEM((_L,), jnp.float32),
            pltpu.VMEM((rows, Wd), jnp.float32),
            pltpu.VMEM((rows, Wd), jnp.float32),
            pltpu.VMEM((_EMB, rows, Wd), jnp.float32),
            pltpu.VMEM((_EMB, rows, Wd), jnp.float32),
            pltpu.SemaphoreType.DMA,
            pltpu.SemaphoreType.DMA,
            pltpu.SemaphoreType.DMA,
            pltpu.SemaphoreType.DMA,
        ],
    )
    def k(f_hbm, w_hbm, m_hbm, out_hbm, w_v, m_v, in0, in1,
          ob0, ob1, is0, is1, os0, os1):
        ins = (in0, in1)
        outs = (ob0, ob1)
        isems = (is0, is1)
        osems = (os0, os1)
        wid = lax.axis_index("s") * _NC + lax.axis_index("c")
        pltpu.sync_copy(w_hbm, w_v)
        pltpu.sync_copy(m_hbm.at[0, pl.ds(0, _L)], m_v)
        m = m_v[...]
        scale = float(_NUM_EMB)
        pln0 = wid * planes_per_tile

        def in_slice(step):
            pln = pln0 + step // pieces_per_plane
            r0 = (step % pieces_per_plane) * rows
            return f_hbm.at[pln, 0, pl.ds(r0, rows), :]

        wshift = Wd.bit_length() - 1
        wmask = Wd - 1

        def compute(b):
            @plsc.parallel_loop(0, piece, _L, unroll=8)
            def _(i):
                r = i >> wshift
                c = i & wmask
                f = ins[b][r, pl.ds(c, _L)]
                ix = ((f - m) * scale).astype(jnp.int32) * _EMB
                for e in range(_EMB):
                    outs[b][e, r, pl.ds(c, _L)] = plsc.load_gather(
                        w_v, [ix + e]
                    )

        pltpu.async_copy(in_slice(0), ins[0], isems[0])
        pltpu.async_copy(in_slice(1), ins[1], isems[1])

        def pair_body(p, _):
            for b in range(2):
                step = p * 2 + b
                pltpu.make_async_copy(in_slice(step), ins[b], isems[b]).wait()

                @pl.when(p > 0)
                def _():
                    pltpu.make_async_copy(
                        out_hbm.at[0, :, pl.ds(0, rows), :], outs[b], osems[b]
                    ).wait()

                compute(b)
                pln = pln0 + step // pieces_per_plane
                r0 = (step % pieces_per_plane) * rows
                dst = out_hbm.at[pln, :, pl.ds(r0, rows), :]
                pltpu.async_copy(outs[b], dst, osems[b])

                @pl.when(step + 2 < nsteps)
                def _():
                    pltpu.async_copy(in_slice(step + 2), ins[b], isems[b])
            return 0

        lax.fori_loop(0, nsteps // 2, pair_body, 0)
        for b in range(2):
            pltpu.make_async_copy(
                out_hbm.at[0, :, pl.ds(0, rows), :], outs[b], osems[b]
            ).wait()

    return k(feature, w_flat, m2d)


def kernel(feature, W):
    B, C, H, Wd = feature.shape
    assert B % _NW == 0
    planes_per_tile = B // _NW
    m2d = _global_min(feature)
    return _sc_lookup(feature, W.reshape(-1), m2d, planes_per_tile, rows=16)


# speedup vs baseline: 1.0810x; 1.0810x over previous
import functools

import jax
import jax.numpy as jnp
from jax import lax
from jax.experimental import pallas as pl
from jax.experimental.pallas import tpu as pltpu
from jax.experimental.pallas import tpu_sc as plsc

_NUM_EMB = 256
_EMB = 3

_info = plsc.get_sparse_core_info()
_NC, _NS, _L = _info.num_cores, _info.num_subcores, _info.num_lanes
_NW = _NC * _NS


def _min_body(x_ref, o_ref):
    @pl.when(pl.program_id(0) == 0)
    def _():
        o_ref[...] = jnp.full_like(o_ref[...], jnp.inf)

    o_ref[...] = jnp.minimum(o_ref[...], jnp.min(x_ref[...]))


def _global_min(feature):
    B, C, H, Wd = feature.shape
    grid = 8
    bb = B // grid
    out = pl.pallas_call(
        _min_body,
        grid=(grid,),
        in_specs=[pl.BlockSpec((bb, C, H, Wd), lambda i: (i, 0, 0, 0))],
        out_specs=pl.BlockSpec((8, 128), lambda i: (0, 0)),
        out_shape=jax.ShapeDtypeStruct((8, 128), jnp.float32),
    )(feature)
    return out


def _sc_lookup(feature, w_flat, m2d, planes_per_tile, rows):
    B, C, H, Wd = feature.shape
    piece = rows * Wd
    pieces_per_plane = H // rows
    nsteps = planes_per_tile * pieces_per_plane
    assert nsteps % 2 == 0
    mesh = plsc.VectorSubcoreMesh(core_axis_name="c", subcore_axis_name="s")

    @functools.partial(
        pl.kernel,
        mesh=mesh,
        compiler_params=pltpu.CompilerParams(needs_layout_passes=False),
        out_type=jax.ShapeDtypeStruct((B, _EMB, H, Wd), jnp.float32),
        scratch_types=[
            pltpu.VMEM((_EMB * _NUM_EMB,), jnp.float32),
            pltpu.VMEM((_L,), jnp.float32),
            pltpu.VMEM((rows, Wd), jnp.float32),
            pltpu.VMEM((rows, Wd), jnp.float32),
            pltpu.VMEM((_EMB, rows, Wd), jnp.float32),
            pltpu.VMEM((_EMB, rows, Wd), jnp.float32),
            pltpu.SemaphoreType.DMA,
            pltpu.SemaphoreType.DMA,
            pltpu.SemaphoreType.DMA,
            pltpu.SemaphoreType.DMA,
        ],
    )
    def k(f_hbm, w_hbm, m_hbm, out_hbm, w_v, m_v, in0, in1,
          ob0, ob1, is0, is1, os0, os1):
        ins = (in0, in1)
        outs = (ob0, ob1)
        isems = (is0, is1)
        osems = (os0, os1)
        wid = lax.axis_index("s") * _NC + lax.axis_index("c")
        pltpu.sync_copy(w_hbm, w_v)
        pltpu.sync_copy(m_hbm.at[0, pl.ds(0, _L)], m_v)
        m = m_v[...]
        scale = float(_NUM_EMB)
        pln0 = wid * planes_per_tile

        def in_slice(step):
            pln = pln0 + step // pieces_per_plane
            r0 = (step % pieces_per_plane) * rows
            return f_hbm.at[pln, 0, pl.ds(r0, rows), :]

        wshift = Wd.bit_length() - 1
        wmask = Wd - 1

        def compute(b):
            @plsc.parallel_loop(0, piece, _L, unroll=8)
            def _(i):
                r = i >> wshift
                c = i & wmask
                f = ins[b][r, pl.ds(c, _L)]
                ix = ((f - m) * scale).astype(jnp.int32) * _EMB
                for e in range(_EMB):
                    outs[b][e, r, pl.ds(c, _L)] = plsc.load_gather(
                        w_v, [ix + e]
                    )

        pltpu.async_copy(in_slice(0), ins[0], isems[0])
        pltpu.async_copy(in_slice(1), ins[1], isems[1])

        def pair_body(p, _):
            for b in range(2):
                step = p * 2 + b
                pltpu.make_async_copy(in_slice(step), ins[b], isems[b]).wait()

                @pl.when(p > 0)
                def _():
                    pltpu.make_async_copy(
                        out_hbm.at[0, :, pl.ds(0, rows), :], outs[b], osems[b]
                    ).wait()

                compute(b)
                pln = pln0 + step // pieces_per_plane
                r0 = (step % pieces_per_plane) * rows
                dst = out_hbm.at[pln, :, pl.ds(r0, rows), :]
                pltpu.async_copy(outs[b], dst, osems[b])

                @pl.when(step + 2 < nsteps)
                def _():
                    pltpu.async_copy(in_slice(step + 2), ins[b], isems[b])
            return 0

        lax.fori_loop(0, nsteps // 2, pair_body, 0)
        for b in range(2):
            pltpu.make_async_copy(
                out_hbm.at[0, :, pl.ds(0, rows), :], outs[b], osems[b]
            ).wait()

    return k(feature, w_flat, m2d)


def kernel(feature, W):
    B, C, H, Wd = feature.shape
    assert B % _NW == 0
    planes_per_tile = B // _NW
    m2d = _global_min(feature)
    return _sc_lookup(feature, W.reshape(-1), m2d, planes_per_tile, rows=16)
